# baseline (speedup 1.0000x reference)
import functools

import jax
import jax.numpy as jnp
from jax import lax
from jax.experimental import pallas as pl
from jax.experimental.pallas import tpu as pltpu
from jax.experimental.pallas import tpu_sc as plsc

NC = 2
NS = 16
C = 64
G = 27
NB = 6
LANES = 16
ZR = 8


def _sc_partials(x, srcr, dstr, kr, n_pad, d_feat):
    rows_per_tile = n_pad // NS
    assert rows_per_tile % ZR == 0 and (G - 3) % 3 == 0

    mesh = plsc.VectorSubcoreMesh(core_axis_name="c", subcore_axis_name="s")

    @functools.partial(
        pl.kernel,
        out_type=jax.ShapeDtypeStruct((NC, n_pad, d_feat), jnp.float32),
        mesh=mesh,
        compiler_params=pltpu.CompilerParams(use_tc_tiling_on_sc=False),
        scratch_types=[
            pltpu.VMEM((G, C), jnp.int32),
            pltpu.VMEM((G, C), jnp.int32),
            pltpu.VMEM((G, C), jnp.float32),
            pltpu.VMEM((3, C, d_feat), jnp.float32),
            pltpu.VMEM((ZR, d_feat), jnp.float32),
            pltpu.VMEM_SHARED((n_pad, d_feat), jnp.float32),
            pltpu.SemaphoreType.DMA((3,)),
            pltpu.SemaphoreType.DMA((3,)),
        ],
    )
    def sc_kernel(x_hbm, src_hbm, dst_hbm, k_hbm, part_hbm,
                  src_v, dst_v, k_v, rows_v, zbuf_v, acc_sh, gsem, ssem):
        c = lax.axis_index("c")
        s = lax.axis_index("s")
        wid = s * NC + c

        zero16 = jnp.zeros((LANES,), jnp.float32)
        def zero_row(i, _):
            for t in range(d_feat // LANES):
                zbuf_v[i, pl.ds(t * LANES, LANES)] = zero16
            return 0
        lax.fori_loop(0, ZR, zero_row, 0)

        def zcopy(t, _):
            pltpu.sync_copy(zbuf_v,
                            acc_sh.at[pl.ds(s * rows_per_tile + t * ZR, ZR)])
            return 0
        lax.fori_loop(0, rows_per_tile // ZR, zcopy, 0)

        plsc.subcore_barrier()

        def wait_g(p, g):
            pltpu.make_async_copy(x_hbm.at[src_v.at[g]], rows_v.at[p],
                                  gsem.at[p]).wait()

        def issue_g(g, q):
            pltpu.async_copy(x_hbm.at[src_v.at[g]], rows_v.at[q], gsem.at[q])

        def issue_s(g, p):
            pltpu.async_copy(rows_v.at[p], acc_sh.at[dst_v.at[g]],
                             ssem.at[p], add=True)

        def wait_s(q, g):
            pltpu.make_async_copy(rows_v.at[q], acc_sh.at[dst_v.at[g]],
                                  ssem.at[q]).wait()

        def scale(g, p):
            def sgroup(q, _):
                kk = k_v[g, pl.ds(q * LANES, LANES)]
                e0 = q * LANES
                for i in range(LANES):
                    kv = kk[i]
                    for t in range(d_feat // LANES):
                        sl = pl.ds(t * LANES, LANES)
                        rows_v[p, e0 + i, sl] = rows_v[p, e0 + i, sl] * kv
                return 0
            lax.fori_loop(0, C // LANES, sgroup, 0)

        def block_body(jj, _):
            blk = wid * NB + jj
            pltpu.sync_copy(src_hbm.at[blk], src_v)
            pltpu.sync_copy(dst_hbm.at[blk], dst_v)
            pltpu.sync_copy(k_hbm.at[blk], k_v)

            issue_g(0, 0)
            issue_g(1, 1)
            wait_g(0, 0)
            issue_g(2, 2)
            scale(0, 0)
            issue_s(0, 0)
            def mstep(u, _):
                for r in range(3):
                    g = 1 + 3 * u + r
                    p = (1 + r) % 3
                    wait_g(p, g)
                    wait_s(r, g - 1)
                    issue_g(g + 2, r)
                    scale(g, p)
                    issue_s(g, p)
                return 0
            lax.fori_loop(0, (G - 3) // 3, mstep, 0)
            for g in (G - 2, G - 1):
                p = g % 3
                wait_g(p, g)
                wait_s((g + 2) % 3, g - 1)
                scale(g, p)
                issue_s(g, p)
            wait_s((G - 1) % 3, G - 1)
            return 0
        lax.fori_loop(0, NB, block_body, 0)

        plsc.subcore_barrier()

        sl = pl.ds(s * rows_per_tile, rows_per_tile)
        pltpu.sync_copy(acc_sh.at[sl], part_hbm.at[c, sl])

    return sc_kernel(x, srcr, dstr, kr)


def _combine(p, b2, n_nodes, d_feat):
    blk = 400
    assert n_nodes % blk == 0

    def body(p_ref, b_ref, o_ref):
        o_ref[...] = p_ref[0] + p_ref[1] + b_ref[...]

    return pl.pallas_call(
        body,
        out_shape=jax.ShapeDtypeStruct((n_nodes, d_feat), jnp.float32),
        grid=(n_nodes // blk,),
        in_specs=[
            pl.BlockSpec((NC, blk, d_feat), lambda i: (0, i, 0)),
            pl.BlockSpec((blk, 1), lambda i: (i, 0)),
        ],
        out_specs=pl.BlockSpec((blk, d_feat), lambda i: (i, 0)),
    )(p, b2)


def kernel(x, edge_index, k, b):
    n_nodes, d_feat = x.shape
    n_edges = edge_index.shape[1]
    nw = NC * NS
    e_pad = nw * NB * G * C
    assert e_pad >= n_edges
    n_pad = ((n_nodes + NS * ZR - 1) // (NS * ZR)) * NS * ZR
    pad = e_pad - n_edges
    src = jnp.pad(edge_index[0], (0, pad))
    pad_dst = n_nodes + (jnp.arange(pad, dtype=jnp.int32) % (n_pad - n_nodes))
    dst = jnp.concatenate([edge_index[1], pad_dst])
    kp = jnp.pad(k, (0, pad))

    srcr = src.reshape(nw * NB, G, C)
    dstr = dst.reshape(nw * NB, G, C)
    kr = kp.reshape(nw * NB, G, C)
    p = _sc_partials(x, srcr, dstr, kr, n_pad, d_feat)
    return _combine(p, b[:, None], n_nodes, d_feat)


# speedup vs baseline: 1.4985x; 1.4985x over previous
import functools

import jax
import jax.numpy as jnp
from jax import lax
from jax.experimental import pallas as pl
from jax.experimental.pallas import tpu as pltpu
from jax.experimental.pallas import tpu_sc as plsc

NC = 2
NS = 16
C = 48
G = 42
NB = 5
LANES = 16
ZR = 8


def _sc_partials(x, srcr, dstr, kr, n_pad, d_feat):
    rows_per_tile = n_pad // NS
    assert rows_per_tile % ZR == 0

    mesh = plsc.VectorSubcoreMesh(core_axis_name="c", subcore_axis_name="s")

    @functools.partial(
        pl.kernel,
        out_type=jax.ShapeDtypeStruct((NC, n_pad, d_feat), jnp.float32),
        mesh=mesh,
        compiler_params=pltpu.CompilerParams(use_tc_tiling_on_sc=False),
        scratch_types=[
            pltpu.VMEM((G, C), jnp.int32),
            pltpu.VMEM((G, C), jnp.int32),
            pltpu.VMEM((G, C), jnp.float32),
            pltpu.VMEM((C, d_feat), jnp.float32),
            pltpu.VMEM((ZR, d_feat), jnp.float32),
            pltpu.VMEM_SHARED((n_pad, d_feat), jnp.float32),
            pltpu.SemaphoreType.DMA,
        ],
    )
    def sc_kernel(x_hbm, src_hbm, dst_hbm, k_hbm, part_hbm,
                  src_v, dst_v, k_v, rows_v, zbuf_v, acc_sh, sem):
        c = lax.axis_index("c")
        s = lax.axis_index("s")
        wid = s * NC + c

        zero16 = jnp.zeros((LANES,), jnp.float32)
        def zero_row(i, _):
            for t in range(d_feat // LANES):
                zbuf_v[i, pl.ds(t * LANES, LANES)] = zero16
            return 0
        lax.fori_loop(0, ZR, zero_row, 0)

        def zcopy(t, _):
            pltpu.sync_copy(zbuf_v,
                            acc_sh.at[pl.ds(s * rows_per_tile + t * ZR, ZR)])
            return 0
        lax.fori_loop(0, rows_per_tile // ZR, zcopy, 0)

        plsc.subcore_barrier()

        def block_body(jj, _):
            blk = wid * NB + jj
            pltpu.sync_copy(src_hbm.at[blk], src_v)
            pltpu.sync_copy(dst_hbm.at[blk], dst_v)
            pltpu.sync_copy(k_hbm.at[blk], k_v)

            def chunk_body(g, _):
                pltpu.async_copy(x_hbm.at[src_v.at[g]], rows_v, sem).wait()

                def scale_group(q, _):
                    kk = k_v[g, pl.ds(q * LANES, LANES)]
                    e0 = q * LANES
                    for i in range(LANES):
                        kv = kk[i]
                        for t in range(d_feat // LANES):
                            sl = pl.ds(t * LANES, LANES)
                            rows_v[e0 + i, sl] = rows_v[e0 + i, sl] * kv
                    return 0
                lax.fori_loop(0, C // LANES, scale_group, 0)

                pltpu.sync_copy(rows_v, acc_sh.at[dst_v.at[g]], add=True)
                return 0
            lax.fori_loop(0, G, chunk_body, 0)
            return 0
        lax.fori_loop(0, NB, block_body, 0)

        plsc.subcore_barrier()

        sl = pl.ds(s * rows_per_tile, rows_per_tile)
        pltpu.sync_copy(acc_sh.at[sl], part_hbm.at[c, sl])

    return sc_kernel(x, srcr, dstr, kr)


def _combine(p, b2, n_nodes, d_feat):
    blk = 400
    assert n_nodes % blk == 0

    def body(p_ref, b_ref, o_ref):
        o_ref[...] = p_ref[0] + p_ref[1] + b_ref[...]

    return pl.pallas_call(
        body,
        out_shape=jax.ShapeDtypeStruct((n_nodes, d_feat), jnp.float32),
        grid=(n_nodes // blk,),
        in_specs=[
            pl.BlockSpec((NC, blk, d_feat), lambda i: (0, i, 0)),
            pl.BlockSpec((blk, 1), lambda i: (i, 0)),
        ],
        out_specs=pl.BlockSpec((blk, d_feat), lambda i: (i, 0)),
    )(p, b2)


def kernel(x, edge_index, k, b):
    n_nodes, d_feat = x.shape
    n_edges = edge_index.shape[1]
    nw = NC * NS
    e_pad = nw * NB * G * C
    assert e_pad >= n_edges
    n_pad = ((n_nodes + NS * ZR - 1) // (NS * ZR)) * NS * ZR
    pad = e_pad - n_edges
    src = jnp.pad(edge_index[0], (0, pad))
    pad_dst = n_nodes + (jnp.arange(pad, dtype=jnp.int32) % (n_pad - n_nodes))
    dst = jnp.concatenate([edge_index[1], pad_dst])
    kp = jnp.pad(k, (0, pad))

    srcr = src.reshape(nw * NB, G, C)
    dstr = dst.reshape(nw * NB, G, C)
    kr = kp.reshape(nw * NB, G, C)
    p = _sc_partials(x, srcr, dstr, kr, n_pad, d_feat)
    return _combine(p, b[:, None], n_nodes, d_feat)


# speedup vs baseline: 2.3472x; 1.5664x over previous
import functools

import jax
import jax.numpy as jnp
from jax import lax
from jax.experimental import pallas as pl
from jax.experimental.pallas import tpu as pltpu
from jax.experimental.pallas import tpu_sc as plsc

NC = 2
NS = 16
C = 80
G = 25
LANES = 16
ZR = 16


def _sc_partials(x, srcr, dstr, kr, n_pad, d_feat, nb):
    rows_per_tile = n_pad // NS
    assert rows_per_tile % ZR == 0

    mesh = plsc.VectorSubcoreMesh(core_axis_name="c", subcore_axis_name="s")

    @functools.partial(
        pl.kernel,
        out_type=jax.ShapeDtypeStruct((NC, n_pad, d_feat), jnp.float32),
        mesh=mesh,
        compiler_params=pltpu.CompilerParams(use_tc_tiling_on_sc=False),
        scratch_types=[
            pltpu.VMEM((G, C), jnp.int32),
            pltpu.VMEM((G, C), jnp.int32),
            pltpu.VMEM((G, C), jnp.float32),
            pltpu.VMEM((C, d_feat), jnp.float32),
            pltpu.VMEM((ZR, d_feat), jnp.float32),
            pltpu.VMEM_SHARED((n_pad, d_feat), jnp.float32),
            pltpu.SemaphoreType.DMA,
        ],
    )
    def sc_kernel(x_hbm, src_hbm, dst_hbm, k_hbm, part_hbm,
                  src_v, dst_v, k_v, rows_v, zbuf_v, acc_sh, sem):
        c = lax.axis_index("c")
        s = lax.axis_index("s")
        wid = s * NC + c

        zero16 = jnp.zeros((LANES,), jnp.float32)
        def zero_row(i, _):
            for t in range(d_feat // LANES):
                zbuf_v[i, pl.ds(t * LANES, LANES)] = zero16
            return 0
        lax.fori_loop(0, ZR, zero_row, 0)

        def zcopy(t, _):
            pltpu.sync_copy(zbuf_v,
                            acc_sh.at[pl.ds(s * rows_per_tile + t * ZR, ZR)])
            return 0
        lax.fori_loop(0, rows_per_tile // ZR, zcopy, 0)

        plsc.subcore_barrier()

        def block_body(jj, _):
            blk = wid * nb + jj
            pltpu.sync_copy(src_hbm.at[blk], src_v)
            pltpu.sync_copy(dst_hbm.at[blk], dst_v)
            pltpu.sync_copy(k_hbm.at[blk], k_v)

            def chunk_body(g, _):
                pltpu.async_copy(x_hbm.at[src_v.at[g]], rows_v, sem).wait()

                def scale_group(q, _):
                    kk = k_v[g, pl.ds(q * LANES, LANES)]
                    e0 = q * LANES
                    for i in range(LANES):
                        kv = kk[i]
                        for t in range(d_feat // LANES):
                            sl = pl.ds(t * LANES, LANES)
                            rows_v[e0 + i, sl] = rows_v[e0 + i, sl] * kv
                    return 0
                lax.fori_loop(0, C // LANES, scale_group, 0)

                pltpu.sync_copy(rows_v, acc_sh.at[dst_v.at[g]], add=True)
                return 0
            lax.fori_loop(0, G, chunk_body, 0)
            return 0
        lax.fori_loop(0, nb, block_body, 0)

        plsc.subcore_barrier()

        sl = pl.ds(s * rows_per_tile, rows_per_tile)
        pltpu.sync_copy(acc_sh.at[sl], part_hbm.at[c, sl])

    return sc_kernel(x, srcr, dstr, kr)


def _combine(p, b2, n_nodes, d_feat):
    blk = 400
    assert n_nodes % blk == 0

    def body(p_ref, b_ref, o_ref):
        o_ref[...] = p_ref[0] + p_ref[1] + b_ref[...]

    return pl.pallas_call(
        body,
        out_shape=jax.ShapeDtypeStruct((n_nodes, d_feat), jnp.float32),
        grid=(n_nodes // blk,),
        in_specs=[
            pl.BlockSpec((NC, blk, d_feat), lambda i: (0, i, 0)),
            pl.BlockSpec((blk, 1), lambda i: (i, 0)),
        ],
        out_specs=pl.BlockSpec((blk, d_feat), lambda i: (i, 0)),
    )(p, b2)


def kernel(x, edge_index, k, b):
    n_nodes, d_feat = x.shape
    n_edges = edge_index.shape[1]
    nw = NC * NS
    assert n_edges % (nw * G * C) == 0
    nb = n_edges // (nw * G * C)

    srcr = edge_index[0].reshape(nw * nb, G, C)
    dstr = edge_index[1].reshape(nw * nb, G, C)
    kr = k.reshape(nw * nb, G, C)

    n_pad = ((n_nodes + NS * ZR - 1) // (NS * ZR)) * NS * ZR
    p = _sc_partials(x, srcr, dstr, kr, n_pad, d_feat, nb)
    return _combine(p, b[:, None], n_nodes, d_feat)
